# Initial kernel scaffold; baseline (speedup 1.0000x reference)
import jax
import jax.numpy as jnp
from jax.experimental import pallas as pl


def kernel(em, edge_index, edge_features, W1, b1, W2, b2, We, be, Wl1, bl1, Wl2, bl2, Wl3, bl3, gx, bx, ge, be2):
    raise NotImplementedError("write your pallas kernel here")


# speedup vs baseline: 2.2920x; 2.2920x over previous
import functools

import jax
import jax.numpy as jnp
from jax import lax
from jax.experimental import pallas as pl
from jax.experimental.pallas import tpu as pltpu
from jax.experimental.pallas import tpu_sc as plsc

_NC = 2
_NS = 16
_L = 16


def _edge_embed_body(ef_ref, wet_ref, be_ref, out_ref):
    out_ref[...] = (
        jnp.dot(ef_ref[...], wet_ref[...], preferred_element_type=jnp.float32)
        + be_ref[...]
    )


def _edge_embed(ef, WeT, be, block_e):
    E, ED = ef.shape
    D = WeT.shape[1]
    grid = (E // block_e,)
    return pl.pallas_call(
        _edge_embed_body,
        grid=grid,
        in_specs=[
            pl.BlockSpec((block_e, ED), lambda i: (i, 0)),
            pl.BlockSpec((ED, D), lambda i: (0, 0)),
            pl.BlockSpec((1, D), lambda i: (0, 0)),
        ],
        out_specs=pl.BlockSpec((block_e, D), lambda i: (i, 0)),
        out_shape=jax.ShapeDtypeStruct((E, D), jnp.float32),
    )(ef, WeT, be.reshape(1, D))


def _sc_aggregate(em, src, dst, edge_emb, chunk):
    N, D = em.shape
    E = src.shape[0]
    nw = _NC * _NS
    epw = E // nw
    nchunk = epw // chunk
    zrows = 128
    npad = -(-N // (_NS * zrows)) * _NS * zrows
    rows_per_sub = npad // _NS
    nz = rows_per_sub // zrows
    mesh = plsc.VectorSubcoreMesh(core_axis_name="c", subcore_axis_name="s")

    @functools.partial(
        pl.kernel,
        out_type=jax.ShapeDtypeStruct((_NC, npad, D), jnp.float32),
        mesh=mesh,
        scratch_types=[
            pltpu.VMEM((chunk,), jnp.int32),
            pltpu.VMEM((chunk,), jnp.int32),
            pltpu.VMEM((chunk, D), jnp.float32),
            pltpu.VMEM((chunk, D), jnp.float32),
            pltpu.VMEM((zrows, D), jnp.float32),
            pltpu.VMEM_SHARED((npad, D), jnp.float32),
            pltpu.SemaphoreType.DMA,
            pltpu.SemaphoreType.DMA,
        ],
    )
    def k(em_hbm, src_hbm, dst_hbm, emb_hbm, out_hbm,
          src_v, dst_v, rows_v, emb_v, zbuf_v, aggr_sh, sem1, sem2):
        cid = lax.axis_index("c")
        sid = lax.axis_index("s")
        wid = cid * _NS + sid

        zero = jnp.zeros((_L,), jnp.float32)

        def zrow_body(j, _):
            for kk in range(D // _L):
                zbuf_v[j, pl.ds(kk * _L, _L)] = zero
            return 0

        lax.fori_loop(0, zrows, zrow_body, 0)

        def zcopy_body(j, _):
            pltpu.sync_copy(
                zbuf_v,
                aggr_sh.at[pl.ds(sid * rows_per_sub + j * zrows, zrows)],
            )
            return 0

        lax.fori_loop(0, nz, zcopy_body, 0)
        plsc.subcore_barrier()

        base_w = wid * epw

        def chunk_body(j, _):
            base = base_w + j * chunk
            pltpu.sync_copy(src_hbm.at[pl.ds(base, chunk)], src_v)
            pltpu.sync_copy(dst_hbm.at[pl.ds(base, chunk)], dst_v)
            cp1 = pltpu.async_copy(em_hbm.at[src_v], rows_v, sem1)
            cp2 = pltpu.async_copy(emb_hbm.at[pl.ds(base, chunk)], emb_v, sem2)
            cp1.wait()
            cp2.wait()

            def erow(r, _):
                for kk in range(D // _L):
                    sl = pl.ds(kk * _L, _L)
                    rows_v[r, sl] = jnp.maximum(rows_v[r, sl] + emb_v[r, sl], 0.0)
                return 0

            lax.fori_loop(0, chunk, erow, 0)
            pltpu.sync_copy(rows_v, aggr_sh.at[dst_v], add=True)
            return 0

        lax.fori_loop(0, nchunk, chunk_body, 0)
        plsc.subcore_barrier()

        def out_body(j, _):
            r0 = sid * rows_per_sub + j * zrows
            pltpu.sync_copy(aggr_sh.at[pl.ds(r0, zrows)], zbuf_v)
            pltpu.sync_copy(zbuf_v, out_hbm.at[cid].at[pl.ds(r0, zrows)])
            return 0

        lax.fori_loop(0, nz, out_body, 0)

    return k(em, src, dst, edge_emb)


def _node_mlp_body(em_ref, ag_ref, w1t_ref, b1_ref, w2t_ref, b2_ref,
                   gx_ref, bx_ref, wat_ref, wbt_ref, bl1_ref,
                   x_ref, a_ref, b_ref):
    n = em_ref.shape[0]
    h = em_ref[...] + ag_ref[0, :n, :] + ag_ref[1, :n, :]
    h = jnp.maximum(
        jnp.dot(h, w1t_ref[...], preferred_element_type=jnp.float32) + b1_ref[...],
        0.0,
    )
    h = jnp.dot(h, w2t_ref[...], preferred_element_type=jnp.float32) + b2_ref[...]
    mu = jnp.mean(h, axis=0, keepdims=True)
    var = jnp.mean((h - mu) ** 2, axis=0, keepdims=True)
    scale = gx_ref[...] * lax.rsqrt(var + 1e-5)
    xe = jnp.maximum((h - mu) * scale + bx_ref[...], 0.0)
    x_ref[...] = xe
    a_ref[...] = (
        jnp.dot(xe, wat_ref[...], preferred_element_type=jnp.float32) + bl1_ref[...]
    )
    b_ref[...] = jnp.dot(xe, wbt_ref[...], preferred_element_type=jnp.float32)


def _node_mlp(em, aggr, W1T, b1, W2T, b2, gx, bx, WaT, WbT, bl1):
    N, D = em.shape
    out_shape = [jax.ShapeDtypeStruct((N, D), jnp.float32)] * 3
    return pl.pallas_call(
        _node_mlp_body,
        out_shape=out_shape,
    )(em, aggr, W1T, b1.reshape(1, D), W2T, b2.reshape(1, D),
      gx.reshape(1, D), bx.reshape(1, D), WaT, WbT, bl1.reshape(1, D))


def _sc_two_gather(A, B, src, dst, chunk):
    N, D = A.shape
    E = src.shape[0]
    nw = _NC * _NS
    epw = E // nw
    nchunk = epw // chunk
    mesh = plsc.VectorSubcoreMesh(core_axis_name="c", subcore_axis_name="s")

    @functools.partial(
        pl.kernel,
        out_type=jax.ShapeDtypeStruct((E, D), jnp.float32),
        mesh=mesh,
        scratch_types=[
            pltpu.VMEM((chunk,), jnp.int32),
            pltpu.VMEM((chunk,), jnp.int32),
            pltpu.VMEM((chunk, D), jnp.float32),
            pltpu.VMEM((chunk, D), jnp.float32),
            pltpu.SemaphoreType.DMA,
            pltpu.SemaphoreType.DMA,
        ],
    )
    def k(a_hbm, b_hbm, src_hbm, dst_hbm, out_hbm,
          src_v, dst_v, arows_v, brows_v, sem1, sem2):
        cid = lax.axis_index("c")
        sid = lax.axis_index("s")
        wid = cid * _NS + sid
        base_w = wid * epw

        def chunk_body(j, _):
            base = base_w + j * chunk
            pltpu.sync_copy(src_hbm.at[pl.ds(base, chunk)], src_v)
            pltpu.sync_copy(dst_hbm.at[pl.ds(base, chunk)], dst_v)
            cp1 = pltpu.async_copy(a_hbm.at[src_v], arows_v, sem1)
            cp2 = pltpu.async_copy(b_hbm.at[dst_v], brows_v, sem2)
            cp1.wait()
            cp2.wait()

            def erow(r, _):
                for kk in range(D // _L):
                    sl = pl.ds(kk * _L, _L)
                    arows_v[r, sl] = arows_v[r, sl] + brows_v[r, sl]
                return 0

            lax.fori_loop(0, chunk, erow, 0)
            pltpu.sync_copy(arows_v, out_hbm.at[pl.ds(base, chunk)])
            return 0

        lax.fori_loop(0, nchunk, chunk_body, 0)

    return k(A, B, src, dst)


def _edge_mlp_body(g_ref, ef_ref, wct_ref, w2t_ref, bl2_ref, w3t_ref, bl3_ref,
                   e3_ref, stats_ref, acc_ref):
    i = pl.program_id(0)
    e1 = jnp.maximum(
        g_ref[...]
        + jnp.dot(ef_ref[...], wct_ref[...], preferred_element_type=jnp.float32),
        0.0,
    )
    e2 = jnp.maximum(
        jnp.dot(e1, w2t_ref[...], preferred_element_type=jnp.float32) + bl2_ref[...],
        0.0,
    )
    e3 = jnp.dot(e2, w3t_ref[...], preferred_element_type=jnp.float32) + bl3_ref[...]
    e3_ref[...] = e3

    @pl.when(i == 0)
    def _():
        acc_ref[...] = jnp.zeros_like(acc_ref)

    acc_ref[0:1, :] += jnp.sum(e3, axis=0, keepdims=True)
    acc_ref[1:2, :] += jnp.sum(e3 * e3, axis=0, keepdims=True)

    @pl.when(i == pl.num_programs(0) - 1)
    def _():
        stats_ref[...] = acc_ref[...]


def _edge_mlp(g, ef, WcT, W2T, bl2, W3T, bl3, block_e):
    E, D = g.shape
    ED = ef.shape[1]
    grid = (E // block_e,)
    return pl.pallas_call(
        _edge_mlp_body,
        grid=grid,
        in_specs=[
            pl.BlockSpec((block_e, D), lambda i: (i, 0)),
            pl.BlockSpec((block_e, ED), lambda i: (i, 0)),
            pl.BlockSpec((ED, D), lambda i: (0, 0)),
            pl.BlockSpec((D, D), lambda i: (0, 0)),
            pl.BlockSpec((1, D), lambda i: (0, 0)),
            pl.BlockSpec((D, D), lambda i: (0, 0)),
            pl.BlockSpec((1, D), lambda i: (0, 0)),
        ],
        out_specs=[
            pl.BlockSpec((block_e, D), lambda i: (i, 0)),
            pl.BlockSpec((2, D), lambda i: (0, 0)),
        ],
        out_shape=[
            jax.ShapeDtypeStruct((E, D), jnp.float32),
            jax.ShapeDtypeStruct((2, D), jnp.float32),
        ],
        scratch_shapes=[pltpu.VMEM((2, D), jnp.float32)],
    )(g, ef, WcT, W2T, bl2.reshape(1, D), W3T, bl3.reshape(1, D))


def _edge_bn_body(e3_ref, s_ref, t_ref, out_ref):
    out_ref[...] = jnp.maximum(e3_ref[...] * s_ref[...] + t_ref[...], 0.0)


def _edge_bn(e3, s, t, block_e):
    E, D = e3.shape
    grid = (E // block_e,)
    return pl.pallas_call(
        _edge_bn_body,
        grid=grid,
        in_specs=[
            pl.BlockSpec((block_e, D), lambda i: (i, 0)),
            pl.BlockSpec((1, D), lambda i: (0, 0)),
            pl.BlockSpec((1, D), lambda i: (0, 0)),
        ],
        out_specs=pl.BlockSpec((block_e, D), lambda i: (i, 0)),
        out_shape=jax.ShapeDtypeStruct((E, D), jnp.float32),
    )(e3, s.reshape(1, D), t.reshape(1, D))


def kernel(em, edge_index, edge_features, W1, b1, W2, b2, We, be,
           Wl1, bl1, Wl2, bl2, Wl3, bl3, gx, bx, ge, be2):
    N, D = em.shape
    E, ED = edge_features.shape
    src = edge_index[0]
    dst = edge_index[1]

    WeT = We.T
    W1T = W1.T
    W2T = W2.T
    WaT = Wl1[:, :D].T
    WbT = Wl1[:, D:2 * D].T
    WcT = Wl1[:, 2 * D:].T
    W2lT = Wl2.T
    W3lT = Wl3.T

    block_e = 2000

    edge_emb = _edge_embed(edge_features, WeT, be, block_e)
    aggr = _sc_aggregate(em, src, dst, edge_emb, chunk=80)
    x_em, A, B = _node_mlp(em, aggr, W1T, b1, W2T, b2, gx, bx, WaT, WbT, bl1)
    g = _sc_two_gather(A, B, src, dst, chunk=80)
    e3, stats = _edge_mlp(g, edge_features, WcT, W2lT, bl2, W3lT, bl3, block_e)

    mu = stats[0] / E
    var = stats[1] / E - mu * mu
    s = ge * lax.rsqrt(var + 1e-5)
    t = be2 - mu * s
    edge_out = _edge_bn(e3, s, t, block_e)

    return (x_em, edge_out)


# speedup vs baseline: 3.2520x; 1.4189x over previous
import functools

import jax
import jax.numpy as jnp
from jax import lax
from jax.experimental import pallas as pl
from jax.experimental.pallas import tpu as pltpu
from jax.experimental.pallas import tpu_sc as plsc

_NC = 2
_NS = 16
_L = 16


def _edge_embed_body(ef_ref, wet_ref, be_ref, out_ref):
    out_ref[...] = (
        jnp.dot(ef_ref[...], wet_ref[...], preferred_element_type=jnp.float32)
        + be_ref[...]
    )


def _edge_embed(ef, WeT, be, block_e):
    E, ED = ef.shape
    D = WeT.shape[1]
    grid = (E // block_e,)
    return pl.pallas_call(
        _edge_embed_body,
        grid=grid,
        in_specs=[
            pl.BlockSpec((block_e, ED), lambda i: (i, 0)),
            pl.BlockSpec((ED, D), lambda i: (0, 0)),
            pl.BlockSpec((1, D), lambda i: (0, 0)),
        ],
        out_specs=pl.BlockSpec((block_e, D), lambda i: (i, 0)),
        out_shape=jax.ShapeDtypeStruct((E, D), jnp.float32),
    )(ef, WeT, be.reshape(1, D))


def _sc_aggregate(em, src, dst, edge_emb, chunk):
    N, D = em.shape
    E = src.shape[0]
    nw = _NC * _NS
    epw = E // nw
    nchunk = epw // chunk
    zrows = chunk
    npad = -(-N // (_NS * zrows)) * _NS * zrows
    rows_per_sub = npad // _NS
    nz = rows_per_sub // zrows
    mesh = plsc.VectorSubcoreMesh(core_axis_name="c", subcore_axis_name="s")

    @functools.partial(
        pl.kernel,
        out_type=jax.ShapeDtypeStruct((_NC, npad, D), jnp.float32),
        mesh=mesh,
        scratch_types=[
            pltpu.VMEM((epw,), jnp.int32),
            pltpu.VMEM((chunk,), jnp.int32),
            pltpu.VMEM((chunk,), jnp.int32),
            pltpu.VMEM((chunk, D), jnp.float32),
            pltpu.VMEM((chunk, D), jnp.float32),
            pltpu.VMEM((chunk, D), jnp.float32),
            pltpu.VMEM((chunk, D), jnp.float32),
            pltpu.VMEM_SHARED((npad, D), jnp.float32),
            pltpu.SemaphoreType.DMA,
            pltpu.SemaphoreType.DMA,
            pltpu.SemaphoreType.DMA,
            pltpu.SemaphoreType.DMA,
            pltpu.SemaphoreType.DMA,
            pltpu.SemaphoreType.DMA,
        ],
    )
    def k(em_hbm, src_hbm1, dst_hbm1, emb_hbm, out_hbm,
          srcs_v, didx0, didx1, rows0, rows1, emb0, emb1, aggr_sh,
          sg0, sg1, se0, se1, sd0, sd1):
        cid = lax.axis_index("c")
        sid = lax.axis_index("s")
        wid = cid * _NS + sid
        base_w = wid * epw

        pltpu.sync_copy(src_hbm1.at[pl.ds(base_w, epw)], srcs_v)

        zero = jnp.zeros((_L,), jnp.float32)

        def zrow_body(j, _):
            for kk in range(D // _L):
                rows0[j, pl.ds(kk * _L, _L)] = zero
            return 0

        lax.fori_loop(0, zrows, zrow_body, 0)

        def zcopy_body(j, _):
            pltpu.sync_copy(
                rows0,
                aggr_sh.at[pl.ds(sid * rows_per_sub + j * zrows, zrows)],
            )
            return 0

        lax.fori_loop(0, nz, zcopy_body, 0)
        plsc.subcore_barrier()

        rows = [rows0, rows1]
        embs = [emb0, emb1]
        didx = [didx0, didx1]
        sg = [sg0, sg1]
        se = [se0, se1]
        sd = [sd0, sd1]

        def start(j, b):
            pltpu.async_copy(
                em_hbm.at[srcs_v.at[pl.ds(j * chunk, chunk)]], rows[b], sg[b])
            pltpu.async_copy(
                emb_hbm.at[pl.ds(base_w + j * chunk, chunk)], embs[b], se[b])
            pltpu.async_copy(
                dst_hbm1.at[pl.ds(base_w + j * chunk, chunk)], didx[b], sd[b])

        def finish(j, b):
            pltpu.make_async_copy(
                em_hbm.at[srcs_v.at[pl.ds(j * chunk, chunk)]], rows[b], sg[b]).wait()
            pltpu.make_async_copy(
                emb_hbm.at[pl.ds(base_w + j * chunk, chunk)], embs[b], se[b]).wait()
            pltpu.make_async_copy(
                dst_hbm1.at[pl.ds(base_w + j * chunk, chunk)], didx[b], sd[b]).wait()

            def erow(r, _):
                for kk in range(D // _L):
                    sl = pl.ds(kk * _L, _L)
                    rows[b][r, sl] = jnp.maximum(rows[b][r, sl] + embs[b][r, sl], 0.0)
                return 0

            lax.fori_loop(0, chunk, erow, 0)
            pltpu.sync_copy(rows[b], aggr_sh.at[didx[b]], add=True)

        start(0, 0)

        def pair_body(i, _):
            j = 2 * i
            start(j + 1, 1)
            finish(j, 0)
            start(j + 2, 0)
            finish(j + 1, 1)
            return 0

        lax.fori_loop(0, (nchunk - 1) // 2, pair_body, 0)
        if nchunk % 2 == 0:
            start(nchunk - 1, 1)
            finish(nchunk - 2, 0)
            finish(nchunk - 1, 1)
        else:
            finish(nchunk - 1, 0)
        plsc.subcore_barrier()

        def out_body(j, _):
            r0 = sid * rows_per_sub + j * zrows
            pltpu.sync_copy(aggr_sh.at[pl.ds(r0, zrows)], rows0)
            pltpu.sync_copy(rows0, out_hbm.at[cid].at[pl.ds(r0, zrows)])
            return 0

        lax.fori_loop(0, nz, out_body, 0)

    return k(em, src, dst, edge_emb)


def _node_mlp_body(em_ref, ag_ref, w1t_ref, b1_ref, w2t_ref, b2_ref,
                   gx_ref, bx_ref, wat_ref, wbt_ref, bl1_ref,
                   x_ref, a_ref, b_ref):
    n = em_ref.shape[0]
    h = em_ref[...] + ag_ref[0, :n, :] + ag_ref[1, :n, :]
    h = jnp.maximum(
        jnp.dot(h, w1t_ref[...], preferred_element_type=jnp.float32) + b1_ref[...],
        0.0,
    )
    h = jnp.dot(h, w2t_ref[...], preferred_element_type=jnp.float32) + b2_ref[...]
    mu = jnp.mean(h, axis=0, keepdims=True)
    var = jnp.mean((h - mu) ** 2, axis=0, keepdims=True)
    scale = gx_ref[...] * lax.rsqrt(var + 1e-5)
    xe = jnp.maximum((h - mu) * scale + bx_ref[...], 0.0)
    x_ref[...] = xe
    a_ref[...] = (
        jnp.dot(xe, wat_ref[...], preferred_element_type=jnp.float32) + bl1_ref[...]
    )
    b_ref[...] = jnp.dot(xe, wbt_ref[...], preferred_element_type=jnp.float32)


def _node_mlp(em, aggr, W1T, b1, W2T, b2, gx, bx, WaT, WbT, bl1):
    N, D = em.shape
    out_shape = [jax.ShapeDtypeStruct((N, D), jnp.float32)] * 3
    return pl.pallas_call(
        _node_mlp_body,
        out_shape=out_shape,
    )(em, aggr, W1T, b1.reshape(1, D), W2T, b2.reshape(1, D),
      gx.reshape(1, D), bx.reshape(1, D), WaT, WbT, bl1.reshape(1, D))


def _sc_two_gather(A, B, src, dst, chunk):
    N, D = A.shape
    E = src.shape[0]
    nw = _NC * _NS
    epw = E // nw
    nchunk = epw // chunk
    mesh = plsc.VectorSubcoreMesh(core_axis_name="c", subcore_axis_name="s")

    @functools.partial(
        pl.kernel,
        out_type=jax.ShapeDtypeStruct((E, D), jnp.float32),
        mesh=mesh,
        scratch_types=[
            pltpu.VMEM((epw,), jnp.int32),
            pltpu.VMEM((epw,), jnp.int32),
            pltpu.VMEM((chunk, D), jnp.float32),
            pltpu.VMEM((chunk, D), jnp.float32),
            pltpu.VMEM((chunk, D), jnp.float32),
            pltpu.VMEM((chunk, D), jnp.float32),
            pltpu.SemaphoreType.DMA,
            pltpu.SemaphoreType.DMA,
            pltpu.SemaphoreType.DMA,
            pltpu.SemaphoreType.DMA,
        ],
    )
    def k(a_hbm, b_hbm, src_hbm1, dst_hbm1, out_hbm,
          srcs_v, dsts_v, arows0, arows1, brows0, brows1,
          sa0, sa1, sb0, sb1):
        cid = lax.axis_index("c")
        sid = lax.axis_index("s")
        wid = cid * _NS + sid
        base_w = wid * epw

        pltpu.sync_copy(src_hbm1.at[pl.ds(base_w, epw)], srcs_v)
        pltpu.sync_copy(dst_hbm1.at[pl.ds(base_w, epw)], dsts_v)

        ar = [arows0, arows1]
        br = [brows0, brows1]
        sa = [sa0, sa1]
        sb = [sb0, sb1]

        def start(j, b):
            pltpu.async_copy(
                a_hbm.at[srcs_v.at[pl.ds(j * chunk, chunk)]], ar[b], sa[b])
            pltpu.async_copy(
                b_hbm.at[dsts_v.at[pl.ds(j * chunk, chunk)]], br[b], sb[b])

        def finish(j, b):
            pltpu.make_async_copy(
                a_hbm.at[srcs_v.at[pl.ds(j * chunk, chunk)]], ar[b], sa[b]).wait()
            pltpu.make_async_copy(
                b_hbm.at[dsts_v.at[pl.ds(j * chunk, chunk)]], br[b], sb[b]).wait()

            def erow(r, _):
                for kk in range(D // _L):
                    sl = pl.ds(kk * _L, _L)
                    ar[b][r, sl] = ar[b][r, sl] + br[b][r, sl]
                return 0

            lax.fori_loop(0, chunk, erow, 0)
            pltpu.sync_copy(ar[b], out_hbm.at[pl.ds(base_w + j * chunk, chunk)])

        start(0, 0)

        def pair_body(i, _):
            j = 2 * i
            start(j + 1, 1)
            finish(j, 0)
            start(j + 2, 0)
            finish(j + 1, 1)
            return 0

        lax.fori_loop(0, (nchunk - 1) // 2, pair_body, 0)
        if nchunk % 2 == 0:
            start(nchunk - 1, 1)
            finish(nchunk - 2, 0)
            finish(nchunk - 1, 1)
        else:
            finish(nchunk - 1, 0)

    return k(A, B, src, dst)


def _edge_mlp_body(g_ref, ef_ref, wct_ref, w2t_ref, bl2_ref, w3t_ref, bl3_ref,
                   e3_ref, stats_ref, acc_ref):
    i = pl.program_id(0)
    e1 = jnp.maximum(
        g_ref[...]
        + jnp.dot(ef_ref[...], wct_ref[...], preferred_element_type=jnp.float32),
        0.0,
    )
    e2 = jnp.maximum(
        jnp.dot(e1, w2t_ref[...], preferred_element_type=jnp.float32) + bl2_ref[...],
        0.0,
    )
    e3 = jnp.dot(e2, w3t_ref[...], preferred_element_type=jnp.float32) + bl3_ref[...]
    e3_ref[...] = e3

    @pl.when(i == 0)
    def _():
        acc_ref[...] = jnp.zeros_like(acc_ref)

    acc_ref[0:1, :] += jnp.sum(e3, axis=0, keepdims=True)
    acc_ref[1:2, :] += jnp.sum(e3 * e3, axis=0, keepdims=True)

    @pl.when(i == pl.num_programs(0) - 1)
    def _():
        stats_ref[...] = acc_ref[...]


def _edge_mlp(g, ef, WcT, W2T, bl2, W3T, bl3, block_e):
    E, D = g.shape
    ED = ef.shape[1]
    grid = (E // block_e,)
    return pl.pallas_call(
        _edge_mlp_body,
        grid=grid,
        in_specs=[
            pl.BlockSpec((block_e, D), lambda i: (i, 0)),
            pl.BlockSpec((block_e, ED), lambda i: (i, 0)),
            pl.BlockSpec((ED, D), lambda i: (0, 0)),
            pl.BlockSpec((D, D), lambda i: (0, 0)),
            pl.BlockSpec((1, D), lambda i: (0, 0)),
            pl.BlockSpec((D, D), lambda i: (0, 0)),
            pl.BlockSpec((1, D), lambda i: (0, 0)),
        ],
        out_specs=[
            pl.BlockSpec((block_e, D), lambda i: (i, 0)),
            pl.BlockSpec((2, D), lambda i: (0, 0)),
        ],
        out_shape=[
            jax.ShapeDtypeStruct((E, D), jnp.float32),
            jax.ShapeDtypeStruct((2, D), jnp.float32),
        ],
        scratch_shapes=[pltpu.VMEM((2, D), jnp.float32)],
    )(g, ef, WcT, W2T, bl2.reshape(1, D), W3T, bl3.reshape(1, D))


def _edge_bn_body(e3_ref, s_ref, t_ref, out_ref):
    out_ref[...] = jnp.maximum(e3_ref[...] * s_ref[...] + t_ref[...], 0.0)


def _edge_bn(e3, s, t, block_e):
    E, D = e3.shape
    grid = (E // block_e,)
    return pl.pallas_call(
        _edge_bn_body,
        grid=grid,
        in_specs=[
            pl.BlockSpec((block_e, D), lambda i: (i, 0)),
            pl.BlockSpec((1, D), lambda i: (0, 0)),
            pl.BlockSpec((1, D), lambda i: (0, 0)),
        ],
        out_specs=pl.BlockSpec((block_e, D), lambda i: (i, 0)),
        out_shape=jax.ShapeDtypeStruct((E, D), jnp.float32),
    )(e3, s.reshape(1, D), t.reshape(1, D))


def kernel(em, edge_index, edge_features, W1, b1, W2, b2, We, be,
           Wl1, bl1, Wl2, bl2, Wl3, bl3, gx, bx, ge, be2):
    N, D = em.shape
    E, ED = edge_features.shape
    src = edge_index[0]
    dst = edge_index[1]

    WeT = We.T
    W1T = W1.T
    W2T = W2.T
    WaT = Wl1[:, :D].T
    WbT = Wl1[:, D:2 * D].T
    WcT = Wl1[:, 2 * D:].T
    W2lT = Wl2.T
    W3lT = Wl3.T

    block_e = 2000

    edge_emb = _edge_embed(edge_features, WeT, be, block_e)
    aggr = _sc_aggregate(em, src, dst, edge_emb, chunk=40)
    x_em, A, B = _node_mlp(em, aggr, W1T, b1, W2T, b2, gx, bx, WaT, WbT, bl1)
    g = _sc_two_gather(A, B, src, dst, chunk=80)
    e3, stats = _edge_mlp(g, edge_features, WcT, W2lT, bl2, W3lT, bl3, block_e)

    mu = stats[0] / E
    var = stats[1] / E - mu * mu
    s = ge * lax.rsqrt(var + 1e-5)
    t = be2 - mu * s
    edge_out = _edge_bn(e3, s, t, block_e)

    return (x_em, edge_out)


# speedup vs baseline: 3.3762x; 1.0382x over previous
import functools

import jax
import jax.numpy as jnp
from jax import lax
from jax.experimental import pallas as pl
from jax.experimental.pallas import tpu as pltpu
from jax.experimental.pallas import tpu_sc as plsc

_NC = 2
_NS = 16
_L = 16


def _edge_embed_body(ef_ref, wet_ref, be_ref, out_ref):
    out_ref[...] = (
        jnp.dot(ef_ref[...], wet_ref[...], preferred_element_type=jnp.float32)
        + be_ref[...]
    )


def _edge_embed(ef, WeT, be, block_e):
    E, ED = ef.shape
    D = WeT.shape[1]
    grid = (E // block_e,)
    return pl.pallas_call(
        _edge_embed_body,
        grid=grid,
        in_specs=[
            pl.BlockSpec((block_e, ED), lambda i: (i, 0)),
            pl.BlockSpec((ED, D), lambda i: (0, 0)),
            pl.BlockSpec((1, D), lambda i: (0, 0)),
        ],
        out_specs=pl.BlockSpec((block_e, D), lambda i: (i, 0)),
        out_shape=jax.ShapeDtypeStruct((E, D), jnp.float32),
    )(ef, WeT, be.reshape(1, D))


def _sc_aggregate(em, src, dst, edge_emb, chunk):
    N, D = em.shape
    E = src.shape[0]
    nw = _NC * _NS
    epw = E // nw
    nchunk = epw // chunk
    zrows = chunk
    npad = -(-N // (_NS * zrows)) * _NS * zrows
    rows_per_sub = npad // _NS
    nz = rows_per_sub // zrows
    mesh = plsc.VectorSubcoreMesh(core_axis_name="c", subcore_axis_name="s")

    @functools.partial(
        pl.kernel,
        out_type=jax.ShapeDtypeStruct((_NC, npad, D), jnp.float32),
        mesh=mesh,
        scratch_types=[
            pltpu.VMEM((epw,), jnp.int32),
            pltpu.VMEM((chunk,), jnp.int32),
            pltpu.VMEM((chunk,), jnp.int32),
            pltpu.VMEM((chunk, D), jnp.float32),
            pltpu.VMEM((chunk, D), jnp.float32),
            pltpu.VMEM((chunk, D), jnp.float32),
            pltpu.VMEM((chunk, D), jnp.float32),
            pltpu.VMEM_SHARED((npad, D), jnp.float32),
            pltpu.SemaphoreType.DMA,
            pltpu.SemaphoreType.DMA,
            pltpu.SemaphoreType.DMA,
            pltpu.SemaphoreType.DMA,
            pltpu.SemaphoreType.DMA,
            pltpu.SemaphoreType.DMA,
        ],
    )
    def k(em_hbm, src_hbm1, dst_hbm1, emb_hbm, out_hbm,
          srcs_v, didx0, didx1, rows0, rows1, emb0, emb1, aggr_sh,
          sg0, sg1, se0, se1, sd0, sd1):
        cid = lax.axis_index("c")
        sid = lax.axis_index("s")
        wid = cid * _NS + sid
        base_w = wid * epw

        pltpu.sync_copy(src_hbm1.at[pl.ds(base_w, epw)], srcs_v)

        zero = jnp.zeros((_L,), jnp.float32)

        def zrow_body(j, _):
            for kk in range(D // _L):
                rows0[j, pl.ds(kk * _L, _L)] = zero
            return 0

        lax.fori_loop(0, zrows, zrow_body, 0)

        def zcopy_body(j, _):
            pltpu.sync_copy(
                rows0,
                aggr_sh.at[pl.ds(sid * rows_per_sub + j * zrows, zrows)],
            )
            return 0

        lax.fori_loop(0, nz, zcopy_body, 0)
        plsc.subcore_barrier()

        rows = [rows0, rows1]
        embs = [emb0, emb1]
        didx = [didx0, didx1]
        sg = [sg0, sg1]
        se = [se0, se1]
        sd = [sd0, sd1]

        def start(j, b):
            pltpu.async_copy(
                em_hbm.at[srcs_v.at[pl.ds(j * chunk, chunk)]], rows[b], sg[b])
            pltpu.async_copy(
                emb_hbm.at[pl.ds(base_w + j * chunk, chunk)], embs[b], se[b])
            pltpu.async_copy(
                dst_hbm1.at[pl.ds(base_w + j * chunk, chunk)], didx[b], sd[b])

        def finish(j, b):
            pltpu.make_async_copy(
                em_hbm.at[srcs_v.at[pl.ds(j * chunk, chunk)]], rows[b], sg[b]).wait()
            pltpu.make_async_copy(
                emb_hbm.at[pl.ds(base_w + j * chunk, chunk)], embs[b], se[b]).wait()
            pltpu.make_async_copy(
                dst_hbm1.at[pl.ds(base_w + j * chunk, chunk)], didx[b], sd[b]).wait()

            def erow(r, _):
                for kk in range(D // _L):
                    sl = pl.ds(kk * _L, _L)
                    rows[b][r, sl] = jnp.maximum(rows[b][r, sl] + embs[b][r, sl], 0.0)
                return 0

            lax.fori_loop(0, chunk, erow, 0)
            pltpu.sync_copy(rows[b], aggr_sh.at[didx[b]], add=True)

        start(0, 0)

        def pair_body(i, _):
            j = 2 * i
            start(j + 1, 1)
            finish(j, 0)
            start(j + 2, 0)
            finish(j + 1, 1)
            return 0

        lax.fori_loop(0, (nchunk - 1) // 2, pair_body, 0)
        if nchunk % 2 == 0:
            start(nchunk - 1, 1)
            finish(nchunk - 2, 0)
            finish(nchunk - 1, 1)
        else:
            finish(nchunk - 1, 0)
        plsc.subcore_barrier()

        def out_body(j, _):
            r0 = sid * rows_per_sub + j * zrows
            pltpu.sync_copy(aggr_sh.at[pl.ds(r0, zrows)], rows0)
            pltpu.sync_copy(rows0, out_hbm.at[cid].at[pl.ds(r0, zrows)])
            return 0

        lax.fori_loop(0, nz, out_body, 0)

    return k(em, src, dst, edge_emb)


def _node_mlp_body(em_ref, ag_ref, w1t_ref, b1_ref, w2t_ref, b2_ref,
                   gx_ref, bx_ref, wat_ref, wbt_ref, bl1_ref,
                   x_ref, a_ref, b_ref):
    n = em_ref.shape[0]
    h = em_ref[...] + ag_ref[0, :n, :] + ag_ref[1, :n, :]
    h = jnp.maximum(
        jnp.dot(h, w1t_ref[...], preferred_element_type=jnp.float32) + b1_ref[...],
        0.0,
    )
    h = jnp.dot(h, w2t_ref[...], preferred_element_type=jnp.float32) + b2_ref[...]
    mu = jnp.mean(h, axis=0, keepdims=True)
    var = jnp.mean((h - mu) ** 2, axis=0, keepdims=True)
    scale = gx_ref[...] * lax.rsqrt(var + 1e-5)
    xe = jnp.maximum((h - mu) * scale + bx_ref[...], 0.0)
    x_ref[...] = xe
    a_ref[...] = (
        jnp.dot(xe, wat_ref[...], preferred_element_type=jnp.float32) + bl1_ref[...]
    )
    b_ref[...] = jnp.dot(xe, wbt_ref[...], preferred_element_type=jnp.float32)


def _node_mlp(em, aggr, W1T, b1, W2T, b2, gx, bx, WaT, WbT, bl1):
    N, D = em.shape
    out_shape = [jax.ShapeDtypeStruct((N, D), jnp.float32)] * 3
    return pl.pallas_call(
        _node_mlp_body,
        out_shape=out_shape,
    )(em, aggr, W1T, b1.reshape(1, D), W2T, b2.reshape(1, D),
      gx.reshape(1, D), bx.reshape(1, D), WaT, WbT, bl1.reshape(1, D))


def _sc_two_gather(A, B, src, dst, chunk):
    N, D = A.shape
    E = src.shape[0]
    nw = _NC * _NS
    epw = E // nw
    nchunk = epw // chunk
    mesh = plsc.VectorSubcoreMesh(core_axis_name="c", subcore_axis_name="s")

    @functools.partial(
        pl.kernel,
        out_type=jax.ShapeDtypeStruct((E, D), jnp.float32),
        mesh=mesh,
        scratch_types=[
            pltpu.VMEM((epw,), jnp.int32),
            pltpu.VMEM((epw,), jnp.int32),
            pltpu.VMEM((chunk, D), jnp.float32),
            pltpu.VMEM((chunk, D), jnp.float32),
            pltpu.VMEM((chunk, D), jnp.float32),
            pltpu.VMEM((chunk, D), jnp.float32),
            pltpu.SemaphoreType.DMA,
            pltpu.SemaphoreType.DMA,
            pltpu.SemaphoreType.DMA,
            pltpu.SemaphoreType.DMA,
        ],
    )
    def k(a_hbm, b_hbm, src_hbm1, dst_hbm1, out_hbm,
          srcs_v, dsts_v, arows0, arows1, brows0, brows1,
          sa0, sa1, sb0, sb1):
        cid = lax.axis_index("c")
        sid = lax.axis_index("s")
        wid = cid * _NS + sid
        base_w = wid * epw

        pltpu.sync_copy(src_hbm1.at[pl.ds(base_w, epw)], srcs_v)
        pltpu.sync_copy(dst_hbm1.at[pl.ds(base_w, epw)], dsts_v)

        ar = [arows0, arows1]
        br = [brows0, brows1]
        sa = [sa0, sa1]
        sb = [sb0, sb1]

        def start(j, b):
            pltpu.async_copy(
                a_hbm.at[srcs_v.at[pl.ds(j * chunk, chunk)]], ar[b], sa[b])
            pltpu.async_copy(
                b_hbm.at[dsts_v.at[pl.ds(j * chunk, chunk)]], br[b], sb[b])

        def finish(j, b):
            pltpu.make_async_copy(
                a_hbm.at[srcs_v.at[pl.ds(j * chunk, chunk)]], ar[b], sa[b]).wait()
            pltpu.make_async_copy(
                b_hbm.at[dsts_v.at[pl.ds(j * chunk, chunk)]], br[b], sb[b]).wait()

            def erow(r, _):
                for kk in range(D // _L):
                    sl = pl.ds(kk * _L, _L)
                    ar[b][r, sl] = ar[b][r, sl] + br[b][r, sl]
                return 0

            lax.fori_loop(0, chunk, erow, 0)
            pltpu.sync_copy(ar[b], out_hbm.at[pl.ds(base_w + j * chunk, chunk)])

        start(0, 0)

        def pair_body(i, _):
            j = 2 * i
            start(j + 1, 1)
            finish(j, 0)
            start(j + 2, 0)
            finish(j + 1, 1)
            return 0

        lax.fori_loop(0, (nchunk - 1) // 2, pair_body, 0)
        if nchunk % 2 == 0:
            start(nchunk - 1, 1)
            finish(nchunk - 2, 0)
            finish(nchunk - 1, 1)
        else:
            finish(nchunk - 1, 0)

    return k(A, B, src, dst)


def _edge_mlp_body(g_ref, ef_ref, wct_ref, w2t_ref, bl2_ref, w3t_ref, bl3_ref,
                   e3_ref, stats_ref, acc_ref):
    i = pl.program_id(0)
    e1 = jnp.maximum(
        g_ref[...]
        + jnp.dot(ef_ref[...], wct_ref[...], preferred_element_type=jnp.float32),
        0.0,
    )
    e2 = jnp.maximum(
        jnp.dot(e1, w2t_ref[...], preferred_element_type=jnp.float32) + bl2_ref[...],
        0.0,
    )
    e3 = jnp.dot(e2, w3t_ref[...], preferred_element_type=jnp.float32) + bl3_ref[...]
    e3_ref[...] = e3.astype(jnp.bfloat16)

    @pl.when(i == 0)
    def _():
        acc_ref[...] = jnp.zeros_like(acc_ref)

    acc_ref[0:1, :] += jnp.sum(e3, axis=0, keepdims=True)
    acc_ref[1:2, :] += jnp.sum(e3 * e3, axis=0, keepdims=True)

    @pl.when(i == pl.num_programs(0) - 1)
    def _():
        stats_ref[...] = acc_ref[...]


def _edge_mlp(g, ef, WcT, W2T, bl2, W3T, bl3, block_e):
    E, D = g.shape
    ED = ef.shape[1]
    grid = (E // block_e,)
    return pl.pallas_call(
        _edge_mlp_body,
        grid=grid,
        in_specs=[
            pl.BlockSpec((block_e, D), lambda i: (i, 0)),
            pl.BlockSpec((block_e, ED), lambda i: (i, 0)),
            pl.BlockSpec((ED, D), lambda i: (0, 0)),
            pl.BlockSpec((D, D), lambda i: (0, 0)),
            pl.BlockSpec((1, D), lambda i: (0, 0)),
            pl.BlockSpec((D, D), lambda i: (0, 0)),
            pl.BlockSpec((1, D), lambda i: (0, 0)),
        ],
        out_specs=[
            pl.BlockSpec((block_e, D), lambda i: (i, 0)),
            pl.BlockSpec((2, D), lambda i: (0, 0)),
        ],
        out_shape=[
            jax.ShapeDtypeStruct((E, D), jnp.bfloat16),
            jax.ShapeDtypeStruct((2, D), jnp.float32),
        ],
        scratch_shapes=[pltpu.VMEM((2, D), jnp.float32)],
    )(g, ef, WcT, W2T, bl2.reshape(1, D), W3T, bl3.reshape(1, D))


def _edge_bn_body(e3_ref, s_ref, t_ref, out_ref):
    e3 = e3_ref[...].astype(jnp.float32)
    out_ref[...] = jnp.maximum(e3 * s_ref[...] + t_ref[...], 0.0)


def _edge_bn(e3, s, t, block_e):
    E, D = e3.shape
    grid = (E // block_e,)
    return pl.pallas_call(
        _edge_bn_body,
        grid=grid,
        in_specs=[
            pl.BlockSpec((block_e, D), lambda i: (i, 0)),
            pl.BlockSpec((1, D), lambda i: (0, 0)),
            pl.BlockSpec((1, D), lambda i: (0, 0)),
        ],
        out_specs=pl.BlockSpec((block_e, D), lambda i: (i, 0)),
        out_shape=jax.ShapeDtypeStruct((E, D), jnp.float32),
    )(e3, s.reshape(1, D), t.reshape(1, D))


def kernel(em, edge_index, edge_features, W1, b1, W2, b2, We, be,
           Wl1, bl1, Wl2, bl2, Wl3, bl3, gx, bx, ge, be2):
    N, D = em.shape
    E, ED = edge_features.shape
    src = edge_index[0]
    dst = edge_index[1]

    WeT = We.T
    W1T = W1.T
    W2T = W2.T
    WaT = Wl1[:, :D].T
    WbT = Wl1[:, D:2 * D].T
    WcT = Wl1[:, 2 * D:].T
    W2lT = Wl2.T
    W3lT = Wl3.T

    block_e = 2000

    edge_emb = _edge_embed(edge_features, WeT, be, block_e)
    aggr = _sc_aggregate(em, src, dst, edge_emb, chunk=40)
    x_em, A, B = _node_mlp(em, aggr, W1T, b1, W2T, b2, gx, bx, WaT, WbT, bl1)
    g = _sc_two_gather(A, B, src, dst, chunk=80)
    e3, stats = _edge_mlp(g, edge_features, WcT, W2lT, bl2, W3lT, bl3, block_e)

    mu = stats[0] / E
    var = stats[1] / E - mu * mu
    s = ge * lax.rsqrt(var + 1e-5)
    t = be2 - mu * s
    edge_out = _edge_bn(e3, s, t, block_e)

    return (x_em, edge_out)
